# baseline (speedup 1.0000x reference)
import functools

import jax
import jax.numpy as jnp
from jax.experimental import pallas as pl
from jax.experimental.pallas import tpu as pltpu

N = 4096
D = 128
R = 2
BM = 128


def _body(feat_ref, adj_ref, adjt_ref, w_ref, b_ref, wt_ref, bt_ref,
          out_ref, g_scr, gt_scr, acc_scr, cnt_scr):
    m = pl.program_id(0)
    r = pl.program_id(1)

    @pl.when(m == 0)
    def _prologue():
        f = feat_ref[...]
        g_scr[r] = jnp.dot(f, w_ref[r],
                           preferred_element_type=jnp.float32).astype(jnp.bfloat16)
        gt_scr[r] = jnp.dot(f, wt_ref[r],
                            preferred_element_type=jnp.float32).astype(jnp.bfloat16)

    a = adj_ref[0]
    at = adjt_ref[0]
    ab = a.astype(jnp.bfloat16)
    atb = at.astype(jnp.bfloat16)

    y = jnp.dot(ab, g_scr[r], preferred_element_type=jnp.float32)
    yt = jnp.dot(atb, gt_scr[r], preferred_element_type=jnp.float32)

    deg = jnp.sum(a, axis=1, keepdims=True).astype(jnp.float32)
    degt = jnp.sum(at, axis=1, keepdims=True).astype(jnp.float32)
    mask = (deg > 0.0).astype(jnp.float32)
    maskt = (degt > 0.0).astype(jnp.float32)

    bw = jnp.dot(b_ref[pl.ds(r, 1), :], w_ref[r],
                 preferred_element_type=jnp.float32)
    bwt = jnp.dot(bt_ref[pl.ds(r, 1), :], wt_ref[r],
                  preferred_element_type=jnp.float32)

    contrib = (y / jnp.maximum(deg, 1.0) + mask * bw
               + yt / jnp.maximum(degt, 1.0) + maskt * bwt)
    cnt = mask + maskt

    @pl.when(r == 0)
    def _init():
        acc_scr[...] = contrib
        cnt_scr[...] = cnt

    @pl.when(r == R - 1)
    def _finish():
        total = acc_scr[...] + contrib
        full = cnt_scr[...] + cnt
        out_ref[...] = total / jnp.where(full == 0.0, 1.0, full)


@jax.jit
def kernel(features, adjacencies, adjacencies_t, w, bias, w_t, bias_t):
    grid = (N // BM, R)
    return pl.pallas_call(
        _body,
        grid=grid,
        in_specs=[
            pl.BlockSpec((N, D), lambda m, r: (0, 0)),
            pl.BlockSpec((1, BM, N), lambda m, r: (r, m, 0)),
            pl.BlockSpec((1, BM, N), lambda m, r: (r, m, 0)),
            pl.BlockSpec((R, D, D), lambda m, r: (0, 0, 0)),
            pl.BlockSpec((R, D), lambda m, r: (0, 0)),
            pl.BlockSpec((R, D, D), lambda m, r: (0, 0, 0)),
            pl.BlockSpec((R, D), lambda m, r: (0, 0)),
        ],
        out_specs=pl.BlockSpec((BM, D), lambda m, r: (m, 0)),
        out_shape=jax.ShapeDtypeStruct((N, D), jnp.float32),
        scratch_shapes=[
            pltpu.VMEM((R, N, D), jnp.bfloat16),
            pltpu.VMEM((R, N, D), jnp.bfloat16),
            pltpu.VMEM((BM, D), jnp.float32),
            pltpu.VMEM((BM, 1), jnp.float32),
        ],
    )(features, adjacencies, adjacencies_t, w, bias, w_t, bias_t)


# speedup vs baseline: 1.1968x; 1.1968x over previous
import functools

import jax
import jax.numpy as jnp
from jax.experimental import pallas as pl
from jax.experimental.pallas import tpu as pltpu

N = 4096
D = 128
R = 2
BM = 256
NH = N // 2


def _body(feat_ref, adj_lo, adj_hi, adjt_lo, adjt_hi,
          w_ref, b_ref, wt_ref, bt_ref,
          out_ref, g_scr, gt_scr, acc_scr, cnt_scr):
    m = pl.program_id(0)
    r = pl.program_id(1)

    @pl.when(m == 0)
    def _prologue():
        f = feat_ref[...]
        g_scr[r] = jnp.dot(f, w_ref[r],
                           preferred_element_type=jnp.float32).astype(jnp.bfloat16)
        gt_scr[r] = jnp.dot(f, wt_ref[r],
                            preferred_element_type=jnp.float32).astype(jnp.bfloat16)

    a_lo = adj_lo[0]
    a_hi = adj_hi[0]
    at_lo = adjt_lo[0]
    at_hi = adjt_hi[0]

    g_lo = g_scr[r, :NH]
    g_hi = g_scr[r, NH:]
    gt_lo = gt_scr[r, :NH]
    gt_hi = gt_scr[r, NH:]

    y = (jnp.dot(a_lo.astype(jnp.bfloat16), g_lo, preferred_element_type=jnp.float32)
         + jnp.dot(a_hi.astype(jnp.bfloat16), g_hi, preferred_element_type=jnp.float32))
    yt = (jnp.dot(at_lo.astype(jnp.bfloat16), gt_lo, preferred_element_type=jnp.float32)
          + jnp.dot(at_hi.astype(jnp.bfloat16), gt_hi, preferred_element_type=jnp.float32))

    deg = (jnp.sum(a_lo, axis=1, keepdims=True)
           + jnp.sum(a_hi, axis=1, keepdims=True)).astype(jnp.float32)
    degt = (jnp.sum(at_lo, axis=1, keepdims=True)
            + jnp.sum(at_hi, axis=1, keepdims=True)).astype(jnp.float32)
    mask = (deg > 0.0).astype(jnp.float32)
    maskt = (degt > 0.0).astype(jnp.float32)

    bw = jnp.dot(b_ref[pl.ds(r, 1), :], w_ref[r],
                 preferred_element_type=jnp.float32)
    bwt = jnp.dot(bt_ref[pl.ds(r, 1), :], wt_ref[r],
                  preferred_element_type=jnp.float32)

    contrib = (y / jnp.maximum(deg, 1.0) + mask * bw
               + yt / jnp.maximum(degt, 1.0) + maskt * bwt)
    cnt = mask + maskt

    @pl.when(r == 0)
    def _init():
        acc_scr[...] = contrib
        cnt_scr[...] = cnt

    @pl.when(r == R - 1)
    def _finish():
        total = acc_scr[...] + contrib
        full = cnt_scr[...] + cnt
        out_ref[...] = total / jnp.where(full == 0.0, 1.0, full)


@jax.jit
def kernel(features, adjacencies, adjacencies_t, w, bias, w_t, bias_t):
    grid = (N // BM, R)
    return pl.pallas_call(
        _body,
        grid=grid,
        in_specs=[
            pl.BlockSpec((N, D), lambda m, r: (0, 0)),
            pl.BlockSpec((1, BM, NH), lambda m, r: (r, m, 0)),
            pl.BlockSpec((1, BM, NH), lambda m, r: (r, m, 1)),
            pl.BlockSpec((1, BM, NH), lambda m, r: (r, m, 0)),
            pl.BlockSpec((1, BM, NH), lambda m, r: (r, m, 1)),
            pl.BlockSpec((R, D, D), lambda m, r: (0, 0, 0)),
            pl.BlockSpec((R, D), lambda m, r: (0, 0)),
            pl.BlockSpec((R, D, D), lambda m, r: (0, 0, 0)),
            pl.BlockSpec((R, D), lambda m, r: (0, 0)),
        ],
        out_specs=pl.BlockSpec((BM, D), lambda m, r: (m, 0)),
        out_shape=jax.ShapeDtypeStruct((N, D), jnp.float32),
        scratch_shapes=[
            pltpu.VMEM((R, N, D), jnp.bfloat16),
            pltpu.VMEM((R, N, D), jnp.bfloat16),
            pltpu.VMEM((BM, D), jnp.float32),
            pltpu.VMEM((BM, 1), jnp.float32),
        ],
    )(features, adjacencies, adjacencies, adjacencies_t, adjacencies_t,
      w, bias, w_t, bias_t)


# speedup vs baseline: 1.2652x; 1.0571x over previous
import jax
import jax.numpy as jnp
from jax.experimental import pallas as pl
from jax.experimental.pallas import tpu as pltpu

N = 4096
D = 128
R = 2
BM = 256


def _body(feat_ref, adj_ref, adjt_ref, w_ref, b_ref, wt_ref, bt_ref,
          out_ref, g_scr, gt_scr):
    m = pl.program_id(0)

    @pl.when(m == 0)
    def _prologue():
        f = feat_ref[...]
        for r in range(R):
            g_scr[r] = jnp.dot(f, w_ref[r],
                               preferred_element_type=jnp.float32).astype(jnp.bfloat16)
            gt_scr[r] = jnp.dot(f, wt_ref[r],
                                preferred_element_type=jnp.float32).astype(jnp.bfloat16)

    acc = jnp.zeros((BM, D), jnp.float32)
    cnt = jnp.zeros((BM, 1), jnp.float32)
    for r in range(R):
        a = adj_ref[r]
        at = adjt_ref[r]
        y = jnp.dot(a.astype(jnp.bfloat16), g_scr[r],
                    preferred_element_type=jnp.float32)
        yt = jnp.dot(at.astype(jnp.bfloat16), gt_scr[r],
                     preferred_element_type=jnp.float32)
        deg = jnp.sum(a, axis=1, keepdims=True).astype(jnp.float32)
        degt = jnp.sum(at, axis=1, keepdims=True).astype(jnp.float32)
        mask = (deg > 0.0).astype(jnp.float32)
        maskt = (degt > 0.0).astype(jnp.float32)
        bw = jnp.dot(b_ref[pl.ds(r, 1), :], w_ref[r],
                     preferred_element_type=jnp.float32)
        bwt = jnp.dot(bt_ref[pl.ds(r, 1), :], wt_ref[r],
                      preferred_element_type=jnp.float32)
        acc = acc + (y / jnp.maximum(deg, 1.0) + mask * bw
                     + yt / jnp.maximum(degt, 1.0) + maskt * bwt)
        cnt = cnt + mask + maskt

    out_ref[...] = acc / jnp.where(cnt == 0.0, 1.0, cnt)


@jax.jit
def kernel(features, adjacencies, adjacencies_t, w, bias, w_t, bias_t):
    grid = (N // BM,)
    return pl.pallas_call(
        _body,
        grid=grid,
        in_specs=[
            pl.BlockSpec((N, D), lambda m: (0, 0)),
            pl.BlockSpec((R, BM, N), lambda m: (0, m, 0)),
            pl.BlockSpec((R, BM, N), lambda m: (0, m, 0)),
            pl.BlockSpec((R, D, D), lambda m: (0, 0, 0)),
            pl.BlockSpec((R, D), lambda m: (0, 0)),
            pl.BlockSpec((R, D, D), lambda m: (0, 0, 0)),
            pl.BlockSpec((R, D), lambda m: (0, 0)),
        ],
        out_specs=pl.BlockSpec((BM, D), lambda m: (m, 0)),
        out_shape=jax.ShapeDtypeStruct((N, D), jnp.float32),
        scratch_shapes=[
            pltpu.VMEM((R, N, D), jnp.bfloat16),
            pltpu.VMEM((R, N, D), jnp.bfloat16),
        ],
    )(features, adjacencies, adjacencies_t, w, bias, w_t, bias_t)
